# baseline (speedup 1.0000x reference)
import functools

import jax
import jax.numpy as jnp
from jax import lax
from jax.experimental import pallas as pl
from jax.experimental.pallas import tpu as pltpu
from jax.experimental.pallas import tpu_sc as plsc

N = 16384
D = 4096
L = 16

_info = plsc.get_sparse_core_info()
NC = _info.num_cores
NS = _info.num_subcores
NW = NC * NS

ROWS_PER_W = N // NW
R = 8
CHUNKS = ROWS_PER_W // R
HD = D // 2
GROUPS_H = HD // L


def _body(x_hbm, idx_hbm, out_hbm, idx_v,
          in0, in1, o0, o1, si0, si1, so0, so1):
    wid = lax.axis_index("s") * NC + lax.axis_index("c")
    base = wid * ROWS_PER_W

    pltpu.sync_copy(idx_hbm, idx_v)

    ins = (in0, in1)
    sis = (si0, si1)
    outs = (o0, o1)
    sos = (so0, so1)

    def start_in(c, b):
        pltpu.async_copy(x_hbm.at[pl.ds(base + c * R, R)], ins[b], sis[b])

    def wait_in(c, b):
        pltpu.make_async_copy(x_hbm.at[pl.ds(base + c * R, R)], ins[b],
                              sis[b]).wait()

    def start_out(c, h):
        pltpu.async_copy(
            outs[h],
            out_hbm.at[pl.ds(base + c * R, R), pl.ds(h * HD, HD)], sos[h])

    def wait_out(c, h):
        pltpu.make_async_copy(
            outs[h],
            out_hbm.at[pl.ds(base + c * R, R), pl.ds(h * HD, HD)],
            sos[h]).wait()

    start_in(0, 0)
    start_in(1, 1)

    def outer(cc, carry):
        for bi in range(2):
            c = 2 * cc + bi
            in_v = ins[bi]
            wait_in(c, bi)
            for h in range(2):
                out_v = outs[h]

                @pl.when(c >= 1)
                def _():
                    wait_out(c - 1, h)

                @plsc.parallel_loop(0, GROUPS_H, unroll=4)
                def group(g):
                    col0 = g * L
                    idx_vec = idx_v[pl.ds(h * HD + col0, L)]
                    for r in range(R):
                        row_ids = jnp.full((L,), r, jnp.int32)
                        vals = plsc.load_gather(in_v, [row_ids, idx_vec])
                        out_v[r, pl.ds(col0, L)] = vals
                start_out(c, h)

            @pl.when(c + 2 < CHUNKS)
            def _():
                start_in(c + 2, bi)

        return carry

    lax.fori_loop(0, CHUNKS // 2, outer, 0)
    wait_out(CHUNKS - 1, 0)
    wait_out(CHUNKS - 1, 1)


def kernel(x, idx):
    idx32 = idx.astype(jnp.int32)
    mesh = plsc.VectorSubcoreMesh(core_axis_name="c", subcore_axis_name="s")
    k = functools.partial(
        pl.kernel,
        mesh=mesh,
        compiler_params=pltpu.CompilerParams(needs_layout_passes=False),
        out_type=jax.ShapeDtypeStruct((N, D), jnp.float32),
        scratch_types=[
            pltpu.VMEM((D,), jnp.int32),
            pltpu.VMEM((R, D), jnp.float32),
            pltpu.VMEM((R, D), jnp.float32),
            pltpu.VMEM((R, HD), jnp.float32),
            pltpu.VMEM((R, HD), jnp.float32),
            pltpu.SemaphoreType.DMA,
            pltpu.SemaphoreType.DMA,
            pltpu.SemaphoreType.DMA,
            pltpu.SemaphoreType.DMA,
        ],
    )(_body)
    return k(x, idx32)
